# baseline (device time: 12835 ns/iter reference)
import jax
import jax.numpy as jnp
from jax import lax
from jax.experimental import pallas as pl
from jax.experimental.pallas import tpu as pltpu

N_DEV = 4
B, SQ, SKV, HQ, DH = 2, 128, 512, 4, 64
SKV_SHARD = SKV // N_DEV
DM = 512
NPAIR = HQ // 2
NBLK = B * NPAIR + 1
SQ_GLOBAL = 32
SCALE = 0.125


def kernel(x, Wq, K_ext, V_ext, Wo):
    kv = jnp.stack([K_ext, V_ext]).astype(jnp.bfloat16) \
        .reshape(2, B, SKV_SHARD, HQ * DH)
    xb = x.astype(jnp.bfloat16)
    wqb = Wq.astype(jnp.bfloat16)
    wob = Wo.astype(jnp.bfloat16)

    def body(x_ref, wq_ref, kv_ref, wo_ref, out_ref,
             y_send, y_all, send_sems, recv_sems):
        my_pos = lax.axis_index("i")

        for s in range(N_DEV - 1):
            y_all[s, :, SQ_GLOBAL:, :] = jnp.zeros(
                (NBLK, SQ - SQ_GLOBAL, 128), jnp.bfloat16)

        qi = lax.broadcasted_iota(jnp.int32, (SQ, SKV_SHARD), 0)
        kj = lax.broadcasted_iota(jnp.int32, (SQ, SKV_SHARD), 1) \
            + my_pos * SKV_SHARD
        mask = (jnp.abs(qi - kj) <= 128) | (kj < 32) | (qi < 32)
        bias = jnp.where(mask, 0.0, -1e9)
        lane = lax.broadcasted_iota(jnp.int32, (SKV_SHARD, 128), 1)

        q_all = lax.dot_general(
            x_ref[...].reshape(B * SQ, DM), wq_ref[...],
            (((1,), (0,)), ((), ())),
            preferred_element_type=jnp.float32,
        )

        sent = []

        HALVES = ((0, 2), (2, NBLK))

        def send_half(sub):
            lo, hi = HALVES[sub]
            n = hi - lo
            for r in (1, 2, 3):
                p = (my_pos + r) % N_DEV
                slot = 3 - r
                full = pltpu.make_async_remote_copy(
                    src_ref=y_send.at[pl.ds(lo, n)],
                    dst_ref=y_all.at[slot, pl.ds(lo, n)],
                    send_sem=send_sems.at[slot, sub],
                    recv_sem=recv_sems.at[slot, sub],
                    device_id=(p,), device_id_type=pl.DeviceIdType.MESH,
                )
                part = pltpu.make_async_remote_copy(
                    src_ref=y_send.at[pl.ds(lo, n), pl.ds(0, SQ_GLOBAL), :],
                    dst_ref=y_all.at[slot, pl.ds(lo, n), pl.ds(0, SQ_GLOBAL), :],
                    send_sem=send_sems.at[slot, sub],
                    recv_sem=recv_sems.at[slot, sub],
                    device_id=(p,), device_id_type=pl.DeviceIdType.MESH,
                )
                sent.append((full, part))

                @pl.when(my_pos < 2)
                def _(full=full):
                    full.start()

                @pl.when(my_pos >= 2)
                def _(part=part):
                    part.start()

        own = [None] * NBLK
        den_tile = jnp.zeros((SQ, 128), jnp.float32)
        for b in range(B):
            k_slab = kv_ref[0, b]
            v_slab = kv_ref[1, b]
            q_b = q_all[b * SQ:(b + 1) * SQ]
            for p in range(NPAIR):
                ws = []
                for h in (2 * p, 2 * p + 1):
                    q_bh = (q_b[:, h * DH:(h + 1) * DH] * SCALE).astype(jnp.bfloat16)
                    sc = lax.dot_general(
                        q_bh, k_slab[:, h * DH:(h + 1) * DH],
                        (((1,), (1,)), ((), ())),
                        preferred_element_type=jnp.float32,
                    ) + bias
                    w = jnp.exp(sc).astype(jnp.bfloat16)
                    ws.append(w)
                    e_col = jnp.where(lane == b * HQ + h, 1.0, 0.0
                                      ).astype(jnp.bfloat16)
                    den_tile = den_tile + lax.dot_general(
                        w, e_col, (((1,), (0,)), ((), ())),
                        preferred_element_type=jnp.float32,
                    )
                pair_slab = v_slab[:, p * 128:(p + 1) * 128]
                ve = jnp.where(lane < DH, pair_slab, 0).astype(jnp.bfloat16)
                vo = jnp.where(lane >= DH, pair_slab, 0).astype(jnp.bfloat16)
                y_pair = lax.dot_general(
                    ws[0], ve, (((1,), (0,)), ((), ())),
                    preferred_element_type=jnp.float32,
                ) + lax.dot_general(
                    ws[1], vo, (((1,), (0,)), ((), ())),
                    preferred_element_type=jnp.float32,
                )
                blk = b * NPAIR + p
                own[blk] = y_pair
                y_send[blk] = y_pair.astype(jnp.bfloat16)

                if blk == 1:
                    barrier = pltpu.get_barrier_semaphore()
                    for r in (1, 2, 3):
                        pl.semaphore_signal(
                            barrier, inc=1,
                            device_id=((my_pos + r) % N_DEV,),
                            device_id_type=pl.DeviceIdType.MESH,
                        )
                    pl.semaphore_wait(barrier, N_DEV - 1)
                    send_half(0)

        own[NBLK - 1] = den_tile
        y_send[NBLK - 1] = den_tile.astype(jnp.bfloat16)
        send_half(1)

        tot = list(own)
        for sub, (lo, hi) in enumerate(HALVES):
            n = hi - lo
            for s in (0, 2, 1):
                origin = (my_pos + s + 1) % N_DEV
                fullw = pltpu.make_async_remote_copy(
                    src_ref=y_send.at[pl.ds(lo, n)],
                    dst_ref=y_all.at[s, pl.ds(lo, n)],
                    send_sem=send_sems.at[s, sub], recv_sem=recv_sems.at[s, sub],
                    device_id=(my_pos,), device_id_type=pl.DeviceIdType.MESH,
                )
                partw = pltpu.make_async_remote_copy(
                    src_ref=y_send.at[pl.ds(lo, n), pl.ds(0, SQ_GLOBAL), :],
                    dst_ref=y_all.at[s, pl.ds(lo, n), pl.ds(0, SQ_GLOBAL), :],
                    send_sem=send_sems.at[s, sub], recv_sem=recv_sems.at[s, sub],
                    device_id=(my_pos,), device_id_type=pl.DeviceIdType.MESH,
                )

                @pl.when(origin < 2)
                def _(fullw=fullw):
                    fullw.wait_recv()

                @pl.when(origin >= 2)
                def _(partw=partw):
                    partw.wait_recv()

                for blk in range(lo, hi):
                    tot[blk] = tot[blk] + y_all[s, blk].astype(jnp.float32)

        tot_den = tot[NBLK - 1]
        lane_sq = lax.broadcasted_iota(jnp.int32, (SQ, 128), 1)
        wo_b = wo_ref[...]
        for b in range(B):
            pctx = []
            for p in range(NPAIR):
                blk = b * NPAIR + p
                d_even = tot_den[:, b * HQ + 2 * p:b * HQ + 2 * p + 1]
                d_odd = tot_den[:, b * HQ + 2 * p + 1:b * HQ + 2 * p + 2]
                divisor = jnp.where(lane_sq < DH, d_even, d_odd)
                pctx.append(tot[blk] / divisor)
            ctx_b = jnp.concatenate(pctx, axis=1).astype(jnp.bfloat16)
            out_ref[b] = lax.dot_general(
                ctx_b, wo_b, (((1,), (0,)), ((), ())),
                preferred_element_type=jnp.float32,
            ).astype(jnp.bfloat16)

        for full, part in sent:
            @pl.when(my_pos < 2)
            def _(d=full):
                d.wait_send()

            @pl.when(my_pos >= 2)
            def _(d=part):
                d.wait_send()

    return pl.pallas_call(
        body,
        out_shape=jax.ShapeDtypeStruct((B, SQ, DM), jnp.bfloat16),
        in_specs=[pl.BlockSpec(memory_space=pltpu.VMEM)] * 4,
        out_specs=pl.BlockSpec(memory_space=pltpu.VMEM),
        scratch_shapes=[
            pltpu.VMEM((NBLK, SQ, 128), jnp.bfloat16),
            pltpu.VMEM((N_DEV - 1, NBLK, SQ, 128), jnp.bfloat16),
            pltpu.SemaphoreType.DMA((N_DEV - 1, 2)),
            pltpu.SemaphoreType.DMA((N_DEV - 1, 2)),
        ],
        compiler_params=pltpu.CompilerParams(collective_id=0),
    )(xb, wqb, kv, wob)


# device time: 12663 ns/iter; 1.0136x vs baseline; 1.0136x over previous
import jax
import jax.numpy as jnp
from jax import lax
from jax.experimental import pallas as pl
from jax.experimental.pallas import tpu as pltpu

N_DEV = 4
B, SQ, SKV, HQ, DH = 2, 128, 512, 4, 64
SKV_SHARD = SKV // N_DEV
DM = 512
NPAIR = HQ // 2
NBLK = B * NPAIR + 1
SQ_GLOBAL = 32
SCALE = 0.125


def kernel(x, Wq, K_ext, V_ext, Wo):
    kr = K_ext.astype(jnp.bfloat16).reshape(B, SKV_SHARD, HQ * DH)
    vr = V_ext.astype(jnp.bfloat16).reshape(B, SKV_SHARD, HQ * DH)
    xb = x.astype(jnp.bfloat16)
    wqb = Wq.astype(jnp.bfloat16)
    wob = Wo.astype(jnp.bfloat16)

    def body(x_ref, wq_ref, k_ref, v_ref, wo_ref, out_ref,
             y_send, y_all, send_sems, recv_sems):
        my_pos = lax.axis_index("i")

        for s in range(N_DEV - 1):
            y_all[s, :, SQ_GLOBAL:, :] = jnp.zeros(
                (NBLK, SQ - SQ_GLOBAL, 128), jnp.bfloat16)

        qi = lax.broadcasted_iota(jnp.int32, (SQ, SKV_SHARD), 0)
        kj = lax.broadcasted_iota(jnp.int32, (SQ, SKV_SHARD), 1) \
            + my_pos * SKV_SHARD
        mask = (jnp.abs(qi - kj) <= 128) | (kj < 32) | (qi < 32)
        bias = jnp.where(mask, 0.0, -1e9)
        lane = lax.broadcasted_iota(jnp.int32, (SKV_SHARD, 128), 1)

        q_all = lax.dot_general(
            x_ref[...].reshape(B * SQ, DM), wq_ref[...],
            (((1,), (0,)), ((), ())),
            preferred_element_type=jnp.float32,
        )

        sent = []

        HALVES = ((0, 2), (2, NBLK))

        def send_half(sub):
            lo, hi = HALVES[sub]
            n = hi - lo
            for r in (1, 2, 3):
                p = (my_pos + r) % N_DEV
                slot = 3 - r
                full = pltpu.make_async_remote_copy(
                    src_ref=y_send.at[pl.ds(lo, n)],
                    dst_ref=y_all.at[slot, pl.ds(lo, n)],
                    send_sem=send_sems.at[slot, sub],
                    recv_sem=recv_sems.at[slot, sub],
                    device_id=(p,), device_id_type=pl.DeviceIdType.MESH,
                )
                part = pltpu.make_async_remote_copy(
                    src_ref=y_send.at[pl.ds(lo, n), pl.ds(0, SQ_GLOBAL), :],
                    dst_ref=y_all.at[slot, pl.ds(lo, n), pl.ds(0, SQ_GLOBAL), :],
                    send_sem=send_sems.at[slot, sub],
                    recv_sem=recv_sems.at[slot, sub],
                    device_id=(p,), device_id_type=pl.DeviceIdType.MESH,
                )
                sent.append((full, part))

                @pl.when(my_pos < 2)
                def _(full=full):
                    full.start()

                @pl.when(my_pos >= 2)
                def _(part=part):
                    part.start()

        own = [None] * NBLK
        den_tile = jnp.zeros((SQ, 128), jnp.float32)
        for b in range(B):
            k_slab = k_ref[b]
            v_slab = v_ref[b]
            q_b = q_all[b * SQ:(b + 1) * SQ]
            for p in range(NPAIR):
                ws = []
                for h in (2 * p, 2 * p + 1):
                    q_bh = (q_b[:, h * DH:(h + 1) * DH] * SCALE).astype(jnp.bfloat16)
                    sc = lax.dot_general(
                        q_bh, k_slab[:, h * DH:(h + 1) * DH],
                        (((1,), (1,)), ((), ())),
                        preferred_element_type=jnp.float32,
                    ) + bias
                    w = jnp.exp(sc).astype(jnp.bfloat16)
                    ws.append(w)
                    e_col = jnp.where(lane == b * HQ + h, 1.0, 0.0
                                      ).astype(jnp.bfloat16)
                    den_tile = den_tile + lax.dot_general(
                        w, e_col, (((1,), (0,)), ((), ())),
                        preferred_element_type=jnp.float32,
                    )
                pair_slab = v_slab[:, p * 128:(p + 1) * 128]
                ve = jnp.where(lane < DH, pair_slab, 0).astype(jnp.bfloat16)
                vo = jnp.where(lane >= DH, pair_slab, 0).astype(jnp.bfloat16)
                y_pair = lax.dot_general(
                    ws[0], ve, (((1,), (0,)), ((), ())),
                    preferred_element_type=jnp.float32,
                ) + lax.dot_general(
                    ws[1], vo, (((1,), (0,)), ((), ())),
                    preferred_element_type=jnp.float32,
                )
                blk = b * NPAIR + p
                own[blk] = y_pair
                y_send[blk] = y_pair.astype(jnp.bfloat16)

                if blk == 1:
                    barrier = pltpu.get_barrier_semaphore()
                    for r in (1, 2, 3):
                        pl.semaphore_signal(
                            barrier, inc=1,
                            device_id=((my_pos + r) % N_DEV,),
                            device_id_type=pl.DeviceIdType.MESH,
                        )
                    pl.semaphore_wait(barrier, N_DEV - 1)
                    send_half(0)

        own[NBLK - 1] = den_tile
        y_send[NBLK - 1] = den_tile.astype(jnp.bfloat16)
        send_half(1)

        tot = list(own)
        for sub, (lo, hi) in enumerate(HALVES):
            n = hi - lo
            for s in (0, 2, 1):
                origin = (my_pos + s + 1) % N_DEV
                fullw = pltpu.make_async_remote_copy(
                    src_ref=y_send.at[pl.ds(lo, n)],
                    dst_ref=y_all.at[s, pl.ds(lo, n)],
                    send_sem=send_sems.at[s, sub], recv_sem=recv_sems.at[s, sub],
                    device_id=(my_pos,), device_id_type=pl.DeviceIdType.MESH,
                )
                partw = pltpu.make_async_remote_copy(
                    src_ref=y_send.at[pl.ds(lo, n), pl.ds(0, SQ_GLOBAL), :],
                    dst_ref=y_all.at[s, pl.ds(lo, n), pl.ds(0, SQ_GLOBAL), :],
                    send_sem=send_sems.at[s, sub], recv_sem=recv_sems.at[s, sub],
                    device_id=(my_pos,), device_id_type=pl.DeviceIdType.MESH,
                )

                @pl.when(origin < 2)
                def _(fullw=fullw):
                    fullw.wait_recv()

                @pl.when(origin >= 2)
                def _(partw=partw):
                    partw.wait_recv()

                for blk in range(lo, hi):
                    tot[blk] = tot[blk] + y_all[s, blk].astype(jnp.float32)

        tot_den = tot[NBLK - 1]
        lane_sq = lax.broadcasted_iota(jnp.int32, (SQ, 128), 1)
        wo_b = wo_ref[...]
        for b in range(B):
            pctx = []
            for p in range(NPAIR):
                blk = b * NPAIR + p
                d_even = tot_den[:, b * HQ + 2 * p:b * HQ + 2 * p + 1]
                d_odd = tot_den[:, b * HQ + 2 * p + 1:b * HQ + 2 * p + 2]
                divisor = jnp.where(lane_sq < DH, d_even, d_odd)
                pctx.append(tot[blk] / divisor)
            ctx_b = jnp.concatenate(pctx, axis=1).astype(jnp.bfloat16)
            out_ref[b] = lax.dot_general(
                ctx_b, wo_b, (((1,), (0,)), ((), ())),
                preferred_element_type=jnp.float32,
            ).astype(jnp.bfloat16)

        for full, part in sent:
            @pl.when(my_pos < 2)
            def _(d=full):
                d.wait_send()

            @pl.when(my_pos >= 2)
            def _(d=part):
                d.wait_send()

    return pl.pallas_call(
        body,
        out_shape=jax.ShapeDtypeStruct((B, SQ, DM), jnp.bfloat16),
        in_specs=[pl.BlockSpec(memory_space=pltpu.VMEM)] * 5,
        out_specs=pl.BlockSpec(memory_space=pltpu.VMEM),
        scratch_shapes=[
            pltpu.VMEM((NBLK, SQ, 128), jnp.bfloat16),
            pltpu.VMEM((N_DEV - 1, NBLK, SQ, 128), jnp.bfloat16),
            pltpu.SemaphoreType.DMA((N_DEV - 1, 2)),
            pltpu.SemaphoreType.DMA((N_DEV - 1, 2)),
        ],
        compiler_params=pltpu.CompilerParams(collective_id=0),
    )(xb, wqb, kr, vr, wob)
